# baseline (speedup 1.0000x reference)
import functools

import jax
import jax.numpy as jnp
import numpy as np
from jax import lax
from jax.experimental import pallas as pl
from jax.experimental.pallas import tpu as pltpu

N_TOKENS = 1024
K_CODES = 1024
D_MODEL = 1024
MASK_TOKEN_ID = K_CODES
CHOICE_TEMPERATURE = 4.5
RATIO = 0.5

_MASK_RATIO = 0.5 * (1.0 + np.cos(np.pi * RATIO))
_TEMPERATURE = float(CHOICE_TEMPERATURE * (1.0 - _MASK_RATIO))
_MASK_RATIO_Z = 0.0 if _MASK_RATIO < 1e-08 else _MASK_RATIO
_MASK_LEN = int(np.floor(512 * _MASK_RATIO_Z))

_KCH = 4
_KW = K_CODES // _KCH


@functools.cache
def _gumbel_const():
    with jax.ensure_compile_time_eval():
        g = jax.random.gumbel(jax.random.key(42), (1, N_TOKENS), jnp.float32)
    return np.asarray(g)


def _tc_body(er_ref, w_ref, b_ref, z_ref, mrow_ref, mcol_ref, grow_ref, gcol_ref,
             zp_ref, msel_ref, conf_ref, p_scr):
    n = N_TOKENS
    k = pl.program_id(0)
    chunk = jnp.dot(er_ref[0:1, :], w_ref[...], preferred_element_type=jnp.float32)
    p_scr[:, pl.ds(k * _KW, _KW)] = chunk + b_ref[...]

    @pl.when(k == _KCH - 1)
    def _():
        p = p_scr[...]
        m = jnp.max(p, axis=1, keepdims=True)
        e = jnp.exp(p - m)
        s = jnp.sum(e, axis=1, keepdims=True)
        q = e / s
        zpp = jnp.max(q, axis=1, keepdims=True)
        kk = lax.broadcasted_iota(jnp.int32, (1, K_CODES), 1)
        am = jnp.min(jnp.where(q == zpp, kk, K_CODES), axis=1, keepdims=True)

        mrow = mrow_ref[...]
        zp_ref[...] = jnp.where(mrow, am, z_ref[...])
        conf_row = jnp.where(mrow, zpp + _TEMPERATURE * grow_ref[...], jnp.inf)
        conf_col = jnp.where(mcol_ref[...], zpp + _TEMPERATURE * gcol_ref[...],
                             jnp.inf)
        conf_ref[...] = conf_row
        ii = lax.broadcasted_iota(jnp.int32, (n, n), 1)
        jj = lax.broadcasted_iota(jnp.int32, (n, n), 0)
        cmp = (conf_col < conf_row) | ((conf_col == conf_row) & (jj < ii))
        rank_row = jnp.sum(cmp.astype(jnp.int32), axis=0, keepdims=True)
        msel_ref[...] = (rank_row < _MASK_LEN) & mrow


def kernel(z_indices, mask_b, mask_num, emb, W, b):
    del mask_num
    g_row = jnp.asarray(_gumbel_const())
    zp, msel, conf = pl.pallas_call(
        _tc_body,
        grid=(_KCH,),
        in_specs=[
            pl.BlockSpec((8, D_MODEL), lambda k: (MASK_TOKEN_ID // 8, 0)),
            pl.BlockSpec((D_MODEL, _KW), lambda k: (0, k)),
            pl.BlockSpec((1, _KW), lambda k: (0, k)),
            pl.BlockSpec((1, N_TOKENS), lambda k: (0, 0)),
            pl.BlockSpec((1, N_TOKENS), lambda k: (0, 0)),
            pl.BlockSpec((N_TOKENS, 1), lambda k: (0, 0)),
            pl.BlockSpec((1, N_TOKENS), lambda k: (0, 0)),
            pl.BlockSpec((N_TOKENS, 1), lambda k: (0, 0)),
        ],
        out_specs=[
            pl.BlockSpec((1, N_TOKENS), lambda k: (0, 0)),
            pl.BlockSpec((1, N_TOKENS), lambda k: (0, 0)),
            pl.BlockSpec((1, N_TOKENS), lambda k: (0, 0)),
        ],
        out_shape=(
            jax.ShapeDtypeStruct((1, N_TOKENS), jnp.int32),
            jax.ShapeDtypeStruct((1, N_TOKENS), jnp.bool_),
            jax.ShapeDtypeStruct((1, N_TOKENS), jnp.float32),
        ),
        scratch_shapes=[pltpu.VMEM((1, K_CODES), jnp.float32)],
    )(emb, W, b.reshape(1, K_CODES), z_indices, mask_b,
      mask_b.reshape(N_TOKENS, 1), g_row, g_row.reshape(N_TOKENS, 1))
    return zp, msel, conf


# speedup vs baseline: 1.0894x; 1.0894x over previous
import functools

import jax
import jax.numpy as jnp
import numpy as np
from jax import lax
from jax.experimental import pallas as pl
from jax.experimental.pallas import tpu as pltpu

N_TOKENS = 1024
K_CODES = 1024
D_MODEL = 1024
MASK_TOKEN_ID = K_CODES
CHOICE_TEMPERATURE = 4.5
RATIO = 0.5

_MASK_RATIO = 0.5 * (1.0 + np.cos(np.pi * RATIO))
_TEMPERATURE = float(CHOICE_TEMPERATURE * (1.0 - _MASK_RATIO))
_MASK_RATIO_Z = 0.0 if _MASK_RATIO < 1e-08 else _MASK_RATIO
_MASK_LEN = int(np.floor(512 * _MASK_RATIO_Z))

_KCH = 4
_KW = K_CODES // _KCH


@functools.cache
def _tg_const():
    with jax.ensure_compile_time_eval():
        g = jax.random.gumbel(jax.random.key(42), (1, N_TOKENS), jnp.float32)
        tg = jnp.float32(_TEMPERATURE) * g
    return np.asarray(tg)


def _tc_body(er_ref, w_ref, b_ref, z_ref, mrow_ref, tgr_ref, tgc_ref,
             zp_ref, msel_ref, conf_ref, p_scr):
    n = N_TOKENS
    k = pl.program_id(0)
    chunk = jnp.dot(er_ref[0:1, :], w_ref[...], preferred_element_type=jnp.float32)
    p_scr[:, pl.ds(k * _KW, _KW)] = chunk + jnp.reshape(b_ref[...], (1, _KW))

    @pl.when(k == _KCH - 1)
    def _():
        p = p_scr[...]
        m = jnp.max(p, axis=1, keepdims=True)
        e = jnp.exp(p - m)
        s = jnp.sum(e, axis=1, keepdims=True)
        q = e / s
        zpp = jnp.max(q, axis=1, keepdims=True)
        kk = lax.broadcasted_iota(jnp.int32, (1, K_CODES), 1)
        am = jnp.min(jnp.where(q == zpp, kk, K_CODES), axis=1, keepdims=True)

        mrow = mrow_ref[...]
        zp_ref[...] = jnp.where(mrow, am, z_ref[...])
        conf_row = jnp.where(mrow, zpp + tgr_ref[...], jnp.inf)
        conf_ref[...] = conf_row

        ii = lax.broadcasted_iota(jnp.int32, (n, n), 1)
        jj = lax.broadcasted_iota(jnp.int32, (n, n), 0)
        eye = ii == jj
        ones_col = jnp.ones((n, 1), jnp.float32)
        mrow_f = jnp.where(mrow, 1.0, 0.0)
        mcol = jnp.dot(jnp.where(eye, mrow_f, 0.0), ones_col,
                       preferred_element_type=jnp.float32) > 0.5
        conf_col = jnp.where(mcol, zpp + tgc_ref[...], jnp.inf)
        cmp = (conf_col < conf_row) | ((conf_col == conf_row) & (jj < ii))
        rank_row = jnp.dot(jnp.ones((1, n), jnp.float32), jnp.where(cmp, 1.0, 0.0),
                           preferred_element_type=jnp.float32)
        msel_ref[...] = (rank_row < float(_MASK_LEN)) & mrow


def kernel(z_indices, mask_b, mask_num, emb, W, b):
    del mask_num
    tg_row = jnp.asarray(_tg_const())
    return pl.pallas_call(
        _tc_body,
        grid=(_KCH,),
        in_specs=[
            pl.BlockSpec((8, D_MODEL), lambda k: (MASK_TOKEN_ID // 8, 0)),
            pl.BlockSpec((D_MODEL, _KW), lambda k: (0, k)),
            pl.BlockSpec((_KW,), lambda k: (k,)),
            pl.BlockSpec((1, N_TOKENS), lambda k: (0, 0)),
            pl.BlockSpec((1, N_TOKENS), lambda k: (0, 0)),
            pl.BlockSpec((1, N_TOKENS), lambda k: (0, 0)),
            pl.BlockSpec((N_TOKENS, 1), lambda k: (0, 0)),
        ],
        out_specs=[
            pl.BlockSpec((1, N_TOKENS), lambda k: (0, 0)),
            pl.BlockSpec((1, N_TOKENS), lambda k: (0, 0)),
            pl.BlockSpec((1, N_TOKENS), lambda k: (0, 0)),
        ],
        out_shape=(
            jax.ShapeDtypeStruct((1, N_TOKENS), jnp.int32),
            jax.ShapeDtypeStruct((1, N_TOKENS), jnp.bool_),
            jax.ShapeDtypeStruct((1, N_TOKENS), jnp.float32),
        ),
        scratch_shapes=[pltpu.VMEM((1, K_CODES), jnp.float32)],
    )(emb, W, b, z_indices, mask_b, tg_row, tg_row.reshape(N_TOKENS, 1))


# speedup vs baseline: 1.2503x; 1.1478x over previous
import functools

import jax
import jax.numpy as jnp
import numpy as np
from jax import lax
from jax.experimental import pallas as pl

N_TOKENS = 1024
K_CODES = 1024
D_MODEL = 1024
MASK_TOKEN_ID = K_CODES
CHOICE_TEMPERATURE = 4.5
RATIO = 0.5

_MASK_RATIO = 0.5 * (1.0 + np.cos(np.pi * RATIO))
_TEMPERATURE = float(CHOICE_TEMPERATURE * (1.0 - _MASK_RATIO))
_MASK_RATIO_Z = 0.0 if _MASK_RATIO < 1e-08 else _MASK_RATIO
_MASK_LEN = int(np.floor(512 * _MASK_RATIO_Z))


@functools.cache
def _tg_const():
    with jax.ensure_compile_time_eval():
        g = jax.random.gumbel(jax.random.key(42), (1, N_TOKENS), jnp.float32)
        tg = jnp.float32(_TEMPERATURE) * g
    return np.asarray(tg)


def _tc_body(er_ref, w_ref, b_ref, z_ref, mrow_ref, tgr_ref, tgc_ref,
             zp_ref, msel_ref, conf_ref):
    n = N_TOKENS
    p = jnp.dot(er_ref[0:1, :], w_ref[...], preferred_element_type=jnp.float32)
    p = p + jnp.reshape(b_ref[...], (1, K_CODES))
    m = jnp.max(p, axis=1, keepdims=True)
    e = jnp.exp(p - m)
    s = jnp.sum(e, axis=1, keepdims=True)
    q = e / s
    zpp = jnp.max(q, axis=1, keepdims=True)
    kk = lax.broadcasted_iota(jnp.int32, (1, K_CODES), 1)
    am = jnp.min(jnp.where(q == zpp, kk, K_CODES), axis=1, keepdims=True)

    mrow = mrow_ref[...]
    zp_ref[...] = jnp.where(mrow, am, z_ref[...])
    conf_row = jnp.where(mrow, zpp + tgr_ref[...], jnp.inf)
    conf_ref[...] = conf_row

    ii = lax.broadcasted_iota(jnp.int32, (n, n), 1)
    jj = lax.broadcasted_iota(jnp.int32, (n, n), 0)
    eye = ii == jj
    ones_col = jnp.ones((n, 1), jnp.float32)
    mrow_f = jnp.where(mrow, 1.0, 0.0)
    mcol = jnp.dot(jnp.where(eye, mrow_f, 0.0), ones_col,
                   preferred_element_type=jnp.float32) > 0.5
    conf_col = jnp.where(mcol, zpp + tgc_ref[...], jnp.inf)
    cmp = (conf_col < conf_row) | ((conf_col == conf_row) & (jj < ii))
    rank_row = jnp.dot(jnp.ones((1, n), jnp.float32), jnp.where(cmp, 1.0, 0.0),
                       preferred_element_type=jnp.float32)
    msel_ref[...] = (rank_row < float(_MASK_LEN)) & mrow


def kernel(z_indices, mask_b, mask_num, emb, W, b):
    del mask_num
    tg_row = jnp.asarray(_tg_const())
    return pl.pallas_call(
        _tc_body,
        grid=(1,),
        in_specs=[
            pl.BlockSpec((8, D_MODEL), lambda k: (MASK_TOKEN_ID // 8, 0)),
            pl.BlockSpec((D_MODEL, K_CODES), lambda k: (0, 0)),
            pl.BlockSpec((K_CODES,), lambda k: (0,)),
            pl.BlockSpec((1, N_TOKENS), lambda k: (0, 0)),
            pl.BlockSpec((1, N_TOKENS), lambda k: (0, 0)),
            pl.BlockSpec((1, N_TOKENS), lambda k: (0, 0)),
            pl.BlockSpec((N_TOKENS, 1), lambda k: (0, 0)),
        ],
        out_specs=[
            pl.BlockSpec((1, N_TOKENS), lambda k: (0, 0)),
            pl.BlockSpec((1, N_TOKENS), lambda k: (0, 0)),
            pl.BlockSpec((1, N_TOKENS), lambda k: (0, 0)),
        ],
        out_shape=(
            jax.ShapeDtypeStruct((1, N_TOKENS), jnp.int32),
            jax.ShapeDtypeStruct((1, N_TOKENS), jnp.bool_),
            jax.ShapeDtypeStruct((1, N_TOKENS), jnp.float32),
        ),
    )(emb, W, b, z_indices, mask_b, tg_row, tg_row.reshape(N_TOKENS, 1))
